# baseline (speedup 1.0000x reference)
import jax
import jax.numpy as jnp
from jax import lax
from jax.experimental import pallas as pl
from jax.experimental.pallas import tpu as pltpu
from jax.experimental.pallas import tpu_sc as plsc

_R = 131072
_W = 128
_NC = 2
_NS = 16
_NW = _NC * _NS
_CHUNK = 128
_NCHUNK = _R // (_NW * _CHUNK)
_NBUF = 6
_K = 3


def _sc_flip(x_hbm, o_hbm, *refs):
    idx = refs[0:_NBUF]
    buf = refs[_NBUF : 2 * _NBUF]
    sg = refs[2 * _NBUF : 3 * _NBUF]
    so = refs[3 * _NBUF : 4 * _NBUF]
    wid = lax.axis_index("s") * _NC + lax.axis_index("c")
    base = wid * _NCHUNK
    lane = lax.iota(jnp.int32, 16)

    def fire_gather(c):
        b = c % _NBUF
        top = (base + c) * _CHUNK + (_CHUNK - 1)
        for j in range(_CHUNK // 16):
            idx[b][pl.ds(j * 16, 16)] = top - j * 16 - lane
        pltpu.make_async_copy(x_hbm.at[idx[b]], buf[b], sg[b]).start()

    def out_copy(c):
        b = c % _NBUF
        return pltpu.make_async_copy(
            buf[b], o_hbm.at[pl.ds((base + c) * _CHUNK, _CHUNK)], so[b]
        )

    for c in range(_K):
        fire_gather(c)
    for c in range(_NCHUNK):
        b = c % _NBUF
        cn = c + _K
        if cn < _NCHUNK:
            if cn >= _NBUF:
                out_copy(cn - _NBUF).wait()
            fire_gather(cn)
        pltpu.make_async_copy(x_hbm.at[idx[b]], buf[b], sg[b]).wait()
        out_copy(c).start()
    for c in range(_NCHUNK - _NBUF, _NCHUNK):
        out_copy(c).wait()


def kernel(x):
    B, C, D, H, W = x.shape
    xr = x.reshape(B * C * D * H, W)
    mesh = plsc.VectorSubcoreMesh(core_axis_name="c", subcore_axis_name="s")
    k = pl.kernel(
        _sc_flip,
        mesh=mesh,
        out_type=jax.ShapeDtypeStruct((_R, _W), jnp.float32),
        scratch_types=(
            [pltpu.VMEM((_CHUNK,), jnp.int32) for _ in range(_NBUF)]
            + [pltpu.VMEM((_CHUNK, _W), jnp.float32) for _ in range(_NBUF)]
            + [pltpu.SemaphoreType.DMA for _ in range(2 * _NBUF)]
        ),
    )
    out = k(xr)
    return out.reshape(B, C, D, H, W)


# speedup vs baseline: 1.3616x; 1.3616x over previous
import jax
import jax.numpy as jnp
from jax.experimental import pallas as pl

_HB = 8
_HH = 64
_NG = _HH // _HB


def _flip_body(x_ref, o_ref):
    for g in range(_NG):
        blk = x_ref[:, (_NG - 1 - g) * _HB : (_NG - g) * _HB, :]
        o_ref[:, g * _HB : (g + 1) * _HB, :] = jnp.concatenate(
            [blk[:, i : i + 1, :] for i in reversed(range(_HB))], axis=1
        )


def kernel(x):
    B, C, D, H, W = x.shape
    L = B * C * D
    xr = x.reshape(L, H, W)
    Lb = 128
    out = pl.pallas_call(
        _flip_body,
        grid=(L // Lb, 2),
        in_specs=[pl.BlockSpec((Lb, _HH, W), lambda l, h: (l, h, 0))],
        out_specs=pl.BlockSpec((Lb, _HH, W), lambda l, h: (l, 1 - h, 0)),
        out_shape=jax.ShapeDtypeStruct((L, H, W), x.dtype),
    )(xr)
    return out.reshape(B, C, D, H, W)


# speedup vs baseline: 1.4263x; 1.0475x over previous
import jax
import jax.numpy as jnp
from jax.experimental import pallas as pl

_HB = 8
_NG = 16


def _flip_body(x_ref, o_ref):
    for g in range(_NG):
        blk = x_ref[:, (_NG - 1 - g) * _HB : (_NG - g) * _HB, :]
        o_ref[:, g * _HB : (g + 1) * _HB, :] = jnp.concatenate(
            [blk[:, i : i + 1, :] for i in reversed(range(_HB))], axis=1
        )


def kernel(x):
    B, C, D, H, W = x.shape
    L = B * C * D
    xr = x.reshape(L, H, W)
    Lb = 128
    out = pl.pallas_call(
        _flip_body,
        grid=(L // Lb,),
        in_specs=[pl.BlockSpec((Lb, H, W), lambda l: (l, 0, 0))],
        out_specs=pl.BlockSpec((Lb, H, W), lambda l: (l, 0, 0)),
        out_shape=jax.ShapeDtypeStruct((L, H, W), x.dtype),
    )(xr)
    return out.reshape(B, C, D, H, W)
